# baseline (speedup 1.0000x reference)
import jax
import jax.numpy as jnp
from jax.experimental import pallas as pl
from jax.experimental.pallas import tpu as pltpu

_B, _S, _D = 4, 8192, 4096
_CH = 4096
_NQ = 6
_IDX = 3
_NCHUNK = _B * _S // _CH


def _select_body(in_hbm, out_ref, bufs, sems):
    chunks = [(b, i) for b in range(_B) for i in range(_S // _CH)]
    copies = [
        pltpu.make_async_copy(
            in_hbm.at[b, pl.ds(i * _CH, _CH), pl.ds(0, 128)],
            bufs.at[k % _NQ],
            sems.at[k % _NQ],
        )
        for k, (b, i) in enumerate(chunks)
    ]
    for k in range(_NQ):
        copies[k].start()
    for k in range(_NCHUNK):
        copies[k].wait()
        vals = bufs[k % _NQ, :, _IDX]
        out_ref[pl.ds(k * (_CH // 128), _CH // 128), :] = vals.reshape(
            _CH // 128, 128
        )
        if k + _NQ < _NCHUNK:
            copies[k + _NQ].start()


@jax.jit
def kernel(input_):
    compact = pl.pallas_call(
        _select_body,
        in_specs=[pl.BlockSpec(memory_space=pl.ANY)],
        out_specs=pl.BlockSpec((_B * _S // 128, 128), lambda: (0, 0)),
        out_shape=jax.ShapeDtypeStruct((_B * _S // 128, 128), jnp.float32),
        scratch_shapes=[
            pltpu.VMEM((_NQ, _CH, 128), jnp.float32),
            pltpu.SemaphoreType.DMA((_NQ,)),
        ],
    )(input_)
    return compact.reshape(_B, _S, 1)


# speedup vs baseline: 1.0360x; 1.0360x over previous
import jax
import jax.numpy as jnp
from jax.experimental import pallas as pl
from jax.experimental.pallas import tpu as pltpu

_B, _S, _D = 4, 8192, 4096
_CH = 4096
_NQ = 6
_IDX = 3
_NCHUNK = _B * _S // _CH


def _select_body(in_hbm, out_ref, bufs, sems):
    chunks = [(b, i) for i in range(_S // _CH) for b in range(_B)]
    copies = [
        pltpu.make_async_copy(
            in_hbm.at[b, pl.ds(i * _CH, _CH), pl.ds(0, 128)],
            bufs.at[k % _NQ],
            sems.at[k % _NQ],
        )
        for k, (b, i) in enumerate(chunks)
    ]
    for k in range(_NQ):
        copies[k].start()
    for k, (b, i) in enumerate(chunks):
        copies[k].wait()
        vals = bufs[k % _NQ, :, _IDX]
        off = (b * _S + i * _CH) // 128
        out_ref[pl.ds(off, _CH // 128), :] = vals.reshape(_CH // 128, 128)
        if k + _NQ < _NCHUNK:
            copies[k + _NQ].start()


@jax.jit
def kernel(input_):
    compact = pl.pallas_call(
        _select_body,
        in_specs=[pl.BlockSpec(memory_space=pl.ANY)],
        out_specs=pl.BlockSpec((_B * _S // 128, 128), lambda: (0, 0)),
        out_shape=jax.ShapeDtypeStruct((_B * _S // 128, 128), jnp.float32),
        scratch_shapes=[
            pltpu.VMEM((_NQ, _CH, 128), jnp.float32),
            pltpu.SemaphoreType.DMA((_NQ,)),
        ],
    )(input_)
    return compact.reshape(_B, _S, 1)
